# Initial kernel scaffold; baseline (speedup 1.0000x reference)
import jax
import jax.numpy as jnp
from jax.experimental import pallas as pl


def kernel(x):
    raise NotImplementedError("write your pallas kernel here")


# speedup vs baseline: 12.2620x; 12.2620x over previous
import functools

import jax
import jax.numpy as jnp
from jax import lax
from jax.experimental import pallas as pl
from jax.experimental.pallas import tpu as pltpu

N = 2048
D = 32
K = 16
BR = 256

NEG_INF = float("-inf")


def _topk_body(x_blk, x_full, out_idx):
    r = pl.program_id(0)
    xb = x_blk[...]
    xf = x_full[...]
    g = jax.lax.dot_general(
        xb, xf, (((1,), (1,)), ((), ())),
        precision=lax.Precision.HIGHEST,
        preferred_element_type=jnp.float32)
    nf = jnp.sum(xf * xf, axis=1)[None, :]
    s = 2.0 * g - nf
    cols = lax.broadcasted_iota(jnp.int32, (BR, N), 1)
    rows = r * BR + lax.broadcasted_iota(jnp.int32, (BR, 1), 0)
    s = jnp.where(cols == rows, NEG_INF, s)
    picked_list = []
    for _ in range(K):
        m = jnp.max(s, axis=1, keepdims=True)
        picked = jnp.min(jnp.where(s == m, cols, N), axis=1, keepdims=True)
        picked_list.append(picked)
        s = jnp.where(cols == picked, NEG_INF, s)
    out_idx[...] = jnp.concatenate(picked_list, axis=1)


def _adj_body(tk_blk, tkT, out_a, out_ahat):
    r = pl.program_id(0)
    tkb = tk_blk[...]
    cols = lax.broadcasted_iota(jnp.int32, (BR, N), 1)
    rows = r * BR + lax.broadcasted_iota(jnp.int32, (BR, 1), 0)
    b = jnp.zeros((BR, N), dtype=jnp.bool_)
    for t in range(K):
        b = b | (tkb[:, t][:, None] == cols)
        b = b | (tkT[t, :][None, :] == rows)
    a = b.astype(jnp.float32)
    rowsum = jnp.sum(a, axis=1, keepdims=True)
    out_a[...] = a
    out_ahat[...] = a * (1.0 / rowsum)


@jax.jit
def kernel(x):
    topk = pl.pallas_call(
        _topk_body,
        grid=(N // BR,),
        in_specs=[
            pl.BlockSpec((BR, D), lambda r: (r, 0)),
            pl.BlockSpec((N, D), lambda r: (0, 0)),
        ],
        out_specs=pl.BlockSpec((BR, K), lambda r: (r, 0)),
        out_shape=jax.ShapeDtypeStruct((N, K), jnp.int32),
    )(x, x)
    tkT = topk.T
    a, ahat = pl.pallas_call(
        _adj_body,
        grid=(N // BR,),
        in_specs=[
            pl.BlockSpec((BR, K), lambda r: (r, 0)),
            pl.BlockSpec((K, N), lambda r: (0, 0)),
        ],
        out_specs=[
            pl.BlockSpec((BR, N), lambda r: (r, 0)),
            pl.BlockSpec((BR, N), lambda r: (r, 0)),
        ],
        out_shape=[
            jax.ShapeDtypeStruct((N, N), jnp.float32),
            jax.ShapeDtypeStruct((N, N), jnp.float32),
        ],
    )(topk, tkT)
    return (a, ahat)
